# Initial kernel scaffold; baseline (speedup 1.0000x reference)
import jax
import jax.numpy as jnp
from jax.experimental import pallas as pl


def kernel(x, edge_index, batch, W1, b1, W2, b2, Wh1, bh1, Wh2, bh2):
    raise NotImplementedError("write your pallas kernel here")


# speedup vs baseline: 11.7840x; 11.7840x over previous
import functools

import jax
import jax.numpy as jnp
from jax import lax
from jax.experimental import pallas as pl
from jax.experimental.pallas import tpu as pltpu
from jax.experimental.pallas import tpu_sc as plsc

_NC = 2
_NS = 16
_NW = _NC * _NS
_L = 16
_G = 64


def _mesh():
    return plsc.VectorSubcoreMesh(core_axis_name="c", subcore_axis_name="s",
                                  num_cores=_NC, num_subcores=_NS)


def _pad_rows(N):
    return -(-N // (8 * _NS)) * (8 * _NS)


def _tc_deg(dst_row, dst_col, N, EB):
    E = dst_row.shape[1]
    HI = -(-N // 128)
    grid = E // EB

    def body(dr_ref, dc_ref, o_ref):
        i = pl.program_id(0)

        @pl.when(i == 0)
        def _init():
            o_ref[...] = jnp.zeros_like(o_ref)

        hi = dr_ref[...] // 128
        lo = dc_ref[...] % 128
        oh_hi = (lax.broadcasted_iota(jnp.int32, (HI, 1), 0) == hi
                 ).astype(jnp.float32)
        oh_lo = (lo == lax.broadcasted_iota(jnp.int32, (1, 128), 1)
                 ).astype(jnp.float32)
        o_ref[...] += jnp.dot(oh_hi, oh_lo,
                              preferred_element_type=jnp.float32)

        @pl.when(i == grid - 1)
        def _finish():
            o_ref[...] = lax.rsqrt(o_ref[...] + 1.0)

    return pl.pallas_call(
        body,
        grid=(grid,),
        in_specs=[pl.BlockSpec((1, EB), lambda i: (0, i)),
                  pl.BlockSpec((EB, 1), lambda i: (i, 0))],
        out_specs=pl.BlockSpec((HI, 128), lambda i: (0, 0)),
        out_shape=jax.ShapeDtypeStruct((HI, 128), jnp.float32),
    )(dst_row, dst_col)


@functools.lru_cache(maxsize=None)
def _make_agg_kernel(N, F, E, K):
    EW = E // _NW
    NCHUNK = EW // K
    NP = _pad_rows(N)
    RT = NP // _NS

    @functools.partial(
        pl.kernel,
        out_type=jax.ShapeDtypeStruct((_NC, NP, F), jnp.float32),
        mesh=_mesh(),
        scratch_types=[
            pltpu.VMEM_SHARED((NP, F), jnp.float32),
            pltpu.VMEM((K,), jnp.int32),
            pltpu.VMEM((K,), jnp.int32),
            pltpu.VMEM((K, F), jnp.float32),
            pltpu.SemaphoreType.DMA,
        ],
    )
    def agg_kernel(y_hbm, src_hbm, dst_hbm, z_hbm, out_hbm,
                   acc, idx_s, idx_d, rows, sem):
        c = lax.axis_index("c")
        s = lax.axis_index("s")
        wid = s * _NC + c

        pltpu.sync_copy(z_hbm.at[pl.ds(s * RT, RT), :],
                        acc.at[pl.ds(s * RT, RT), :])
        plsc.subcore_barrier()

        def chunk(ci, carry):
            base = wid * EW + ci * K
            pltpu.sync_copy(src_hbm.at[pl.ds(base, K)], idx_s)
            pltpu.sync_copy(dst_hbm.at[pl.ds(base, K)], idx_d)
            pltpu.async_copy(y_hbm.at[idx_s], rows, sem).wait()
            pltpu.sync_copy(rows, acc.at[idx_d], add=True)
            return carry
        lax.fori_loop(0, NCHUNK, chunk, 0)

        plsc.subcore_barrier()
        pltpu.sync_copy(acc.at[pl.ds(s * RT, RT), :],
                        out_hbm.at[c, pl.ds(s * RT, RT), :])

    return agg_kernel


def _tc_first(dis, x, W):
    N, F = x.shape
    H = W.shape[1]

    def body(dis_ref, x_ref, w_ref, y_ref):
        xw = jnp.dot(x_ref[...], w_ref[...],
                     preferred_element_type=jnp.float32,
                     precision=lax.Precision.HIGHEST)
        y_ref[...] = dis_ref[...] * xw

    return pl.pallas_call(
        body,
        out_shape=jax.ShapeDtypeStruct((N, H), jnp.float32),
    )(dis, x, W)


def _tc_mid(sp, y, dis, b, W):
    N, H = y.shape

    def body(sp_ref, y_ref, dis_ref, b_ref, w_ref, o_ref):
        sagg = sp_ref[0][:N] + sp_ref[1][:N] + y_ref[...]
        h = jnp.maximum(dis_ref[...] * sagg + b_ref[...], 0.0)
        hw = jnp.dot(h, w_ref[...],
                     preferred_element_type=jnp.float32,
                     precision=lax.Precision.HIGHEST)
        o_ref[...] = dis_ref[...] * hw

    return pl.pallas_call(
        body,
        out_shape=jax.ShapeDtypeStruct((N, W.shape[1]), jnp.float32),
    )(sp, y, dis, b, W)


def _tc_final(sp, y, dis, b, batch2d, Wh1, bh1, Wh2, bh2):
    N, H = y.shape

    def body(sp_ref, y_ref, dis_ref, b_ref, bt_ref,
             w1_ref, b1_ref, w2_ref, b2_ref, o_ref):
        h2 = jnp.maximum(
            dis_ref[...] * (sp_ref[0][:N] + sp_ref[1][:N] + y_ref[...])
            + b_ref[...],
            0.0)
        gids = lax.broadcasted_iota(jnp.int32, (_G, 1), 0)
        oh = (gids == bt_ref[...]).astype(jnp.float32)
        cnt = jnp.sum(oh, axis=1, keepdims=True)
        summ = jnp.dot(oh, h2,
                       preferred_element_type=jnp.float32,
                       precision=lax.Precision.HIGHEST)
        pooled = summ / jnp.maximum(cnt, 1.0)
        t = jnp.maximum(
            jnp.dot(pooled, w1_ref[...],
                    preferred_element_type=jnp.float32,
                    precision=lax.Precision.HIGHEST) + b1_ref[...],
            0.0)
        o_ref[...] = jnp.dot(t, w2_ref[...],
                             preferred_element_type=jnp.float32,
                             precision=lax.Precision.HIGHEST) + b2_ref[...]

    return pl.pallas_call(
        body,
        out_shape=jax.ShapeDtypeStruct((_G, 1), jnp.float32),
    )(sp, y, dis, b, batch2d, Wh1, bh1, Wh2, bh2)


def kernel(x, edge_index, batch, W1, b1, W2, b2, Wh1, bh1, Wh2, bh2):
    N, F = x.shape
    E = edge_index.shape[1]
    K = 80

    src = edge_index[0]
    dst = edge_index[1]

    dis_mat = _tc_deg(dst.reshape(1, E), dst.reshape(E, 1), N, 6400)
    dis = dis_mat.reshape(-1, 1)[:N]
    y1 = _tc_first(dis, x.astype(jnp.float32), W1)
    agg = _make_agg_kernel(N, F, E, K)
    z = jnp.zeros((_pad_rows(N), F), jnp.float32)
    s1p = agg(y1, src, dst, z)
    y2 = _tc_mid(s1p, y1, dis, b1, W2)
    s2p = agg(y2, src, dst, z)
    out = _tc_final(s2p, y2, dis, b2,
                    batch.reshape(1, N).astype(jnp.int32),
                    Wh1, bh1, Wh2, bh2)
    return out
